# Initial kernel scaffold; baseline (speedup 1.0000x reference)
import jax
import jax.numpy as jnp
from jax.experimental import pallas as pl


def kernel(states, actions, trs, ccs, k_sa_W, k_sa_b, k_tr_W1, k_tr_b1, k_tr_W2, k_enc_W, k_enc_b, k_dec_W1, k_dec_b1, k_dec_W2, l_sa_W, l_sa_b, l_tr_W1, l_tr_b1, l_tr_W2, l_enc_W1, l_enc_b1, l_enc_W2, gcn_W, gcn_b):
    raise NotImplementedError("write your pallas kernel here")


# speedup vs baseline: 1.1734x; 1.1734x over previous
import jax
import jax.numpy as jnp
from jax.experimental import pallas as pl
from jax.experimental.pallas import tpu as pltpu

_A, _B, _SD, _AD, _H = 8, 4096, 112, 16, 128
_IDIM = _SD + _AD
_SPARSE = 0.05
_TB = 512
_NT = _B // _TB
_F32 = jnp.float32


def _leaky(x):
    return jnp.where(x >= 0, x, 0.01 * x)


def _dot(a, b):
    return jnp.dot(a, b, preferred_element_type=_F32)


def _fused(inps_ref, cc_ref, trs_ref,
           k_sa_W_ref, k_sa_b_ref, k_trW1_ref, k_trb1_ref, k_trW2_ref,
           k_enc_W_ref, k_enc_b_ref, k_dec_W1_ref, k_dec_b1_ref, k_dec_W2_ref,
           l_sa_W_ref, l_sa_b_ref, l_trW1_ref, l_trb1_ref, l_trW2_ref,
           l_enc_W1_ref, l_enc_b1_ref, l_enc_W2_ref, gcn_W_ref, gcn_b_ref,
           out_ref,
           sum_sc, sq_sc, m_sc, s_sc, kb2_sc, lb2_sc, xlin_sc):
    p = pl.program_id(0)
    t = pl.program_id(1)

    @pl.when(p == 0)
    def _stats():
        x = inps_ref[...]
        ssum = jnp.sum(x, axis=1)
        ssq = jnp.sum(x * x, axis=1)

        @pl.when(t == 0)
        def _():
            sum_sc[...] = ssum
            sq_sc[...] = ssq

        @pl.when(t > 0)
        def _():
            sum_sc[...] = sum_sc[...] + ssum
            sq_sc[...] = sq_sc[...] + ssq

    @pl.when((p == 0) & (t == _NT - 1))
    def _finalize():
        m = sum_sc[...] * (1.0 / _B)
        var = sq_sc[...] * (1.0 / _B) - m * m
        m_sc[...] = m
        s_sc[...] = jax.lax.rsqrt(var + 1e-5)
        trs_col = trs_ref[...]
        tvec = _leaky(trs_col * k_trW1_ref[...] + k_trb1_ref[...])
        t2vec = _leaky(trs_col * l_trW1_ref[...] + l_trb1_ref[...])
        for a in range(_A):
            ktr = _leaky(_dot(tvec[a:a + 1, :], k_trW2_ref[a]))
            kb2_sc[a:a + 1, :] = (_dot(ktr, k_enc_W_ref[a, _H:, :])
                                  + k_enc_b_ref[a:a + 1, :])
            ltr = _leaky(_dot(t2vec[a:a + 1, :], l_trW2_ref[a]))
            lb2_sc[a:a + 1, :] = (_dot(ltr, l_enc_W1_ref[a, _H:, :])
                                  + l_enc_b1_ref[a:a + 1, :])

    @pl.when(p == 1)
    def _compute():
        x = inps_ref[...]
        m = m_sc[...]
        s = s_sc[...]
        lams = []
        for a in range(_A):
            xa = (x[a] - m[a:a + 1, :]) * s[a:a + 1, :]
            ksa = _leaky(_dot(xa, k_sa_W_ref[a]) + k_sa_b_ref[a:a + 1, :])
            kenc = _leaky(_dot(ksa, k_enc_W_ref[a, :_H, :]) + kb2_sc[a:a + 1, :])
            xlin_sc[a, :, :] = _dot(kenc, gcn_W_ref[...]) + gcn_b_ref[...]
            lsa = _leaky(_dot(xa, l_sa_W_ref[a]) + l_sa_b_ref[a:a + 1, :])
            e1 = _leaky(_dot(lsa, l_enc_W1_ref[a, :_H, :]) + lb2_sc[a:a + 1, :])
            lams.append(_leaky(_dot(e1, l_enc_W2_ref[a])))

        cc = cc_ref[...]
        lane = jax.lax.broadcasted_iota(jnp.int32, (_TB, _A * _A), 1)
        isdiag = (lane % (_A + 1)) == 0
        mask = jnp.where((cc >= _SPARSE) | isdiag, 1.0, 0.0)
        w = mask * cc
        deg = mask[:, 0:_A]
        for i in range(1, _A):
            deg = deg + mask[:, i * _A:(i + 1) * _A]
        dis = jax.lax.rsqrt(deg)
        ys = [xlin_sc[i] * dis[:, i:i + 1] for i in range(_A)]
        cols = []
        for j in range(_A):
            acc = w[:, j:j + 1] * ys[0]
            for i in range(1, _A):
                acc = acc + w[:, i * _A + j:i * _A + j + 1] * ys[i]
            outj = acc * dis[:, j:j + 1]
            d1 = _leaky(_dot(outj, k_dec_W1_ref[j]) + k_dec_b1_ref[j:j + 1, :])
            kk = _leaky(_dot(d1, k_dec_W2_ref[j]))
            cols.append(lams[j] * kk)
        out_ref[...] = jnp.concatenate(cols, axis=1)


def kernel(states, actions, trs, ccs, k_sa_W, k_sa_b, k_tr_W1, k_tr_b1,
           k_tr_W2, k_enc_W, k_enc_b, k_dec_W1, k_dec_b1, k_dec_W2,
           l_sa_W, l_sa_b, l_tr_W1, l_tr_b1, l_tr_W2, l_enc_W1, l_enc_b1,
           l_enc_W2, gcn_W, gcn_b):
    inps = jnp.concatenate([states, actions], axis=-1)
    cc2 = ccs.reshape(_B, _A * _A)
    trs_col = trs.reshape(_A, 1)
    k_trW1 = k_tr_W1.reshape(_A, _H)
    l_trW1 = l_tr_W1.reshape(_A, _H)
    gcn_b2 = gcn_b.reshape(1, _H)

    def fixed(ndim):
        return lambda p, t: (0,) * ndim

    in_specs = [
        pl.BlockSpec((_A, _TB, _IDIM), lambda p, t: (0, t, 0)),
        pl.BlockSpec((_TB, _A * _A), lambda p, t: (t, 0)),
        pl.BlockSpec((_A, 1), fixed(2)),
        pl.BlockSpec((_A, _IDIM, _H), fixed(3)),
        pl.BlockSpec((_A, _H), fixed(2)),
        pl.BlockSpec((_A, _H), fixed(2)),
        pl.BlockSpec((_A, _H), fixed(2)),
        pl.BlockSpec((_A, _H, _H), fixed(3)),
        pl.BlockSpec((_A, 2 * _H, _H), fixed(3)),
        pl.BlockSpec((_A, _H), fixed(2)),
        pl.BlockSpec((_A, _H, _H), fixed(3)),
        pl.BlockSpec((_A, _H), fixed(2)),
        pl.BlockSpec((_A, _H, 1), fixed(3)),
        pl.BlockSpec((_A, _IDIM, _H), fixed(3)),
        pl.BlockSpec((_A, _H), fixed(2)),
        pl.BlockSpec((_A, _H), fixed(2)),
        pl.BlockSpec((_A, _H), fixed(2)),
        pl.BlockSpec((_A, _H, _H), fixed(3)),
        pl.BlockSpec((_A, 2 * _H, _H), fixed(3)),
        pl.BlockSpec((_A, _H), fixed(2)),
        pl.BlockSpec((_A, _H, 1), fixed(3)),
        pl.BlockSpec((_H, _H), fixed(2)),
        pl.BlockSpec((1, _H), fixed(2)),
    ]

    out = pl.pallas_call(
        _fused,
        grid=(2, _NT),
        in_specs=in_specs,
        out_specs=pl.BlockSpec((_TB, _A), lambda p, t: (t, 0)),
        out_shape=jax.ShapeDtypeStruct((_B, _A), _F32),
        scratch_shapes=[
            pltpu.VMEM((_A, _IDIM), _F32),
            pltpu.VMEM((_A, _IDIM), _F32),
            pltpu.VMEM((_A, _IDIM), _F32),
            pltpu.VMEM((_A, _IDIM), _F32),
            pltpu.VMEM((_A, _H), _F32),
            pltpu.VMEM((_A, _H), _F32),
            pltpu.VMEM((_A, _TB, _H), _F32),
        ],
    )(inps, cc2, trs_col, k_sa_W, k_sa_b, k_trW1, k_tr_b1, k_tr_W2,
      k_enc_W, k_enc_b, k_dec_W1, k_dec_b1, k_dec_W2,
      l_sa_W, l_sa_b, l_trW1, l_tr_b1, l_tr_W2,
      l_enc_W1, l_enc_b1, l_enc_W2, gcn_W, gcn_b2)
    return out
